# initial kernel scaffold (unmeasured)
import jax
import jax.numpy as jnp
from jax import lax
from jax.experimental import pallas as pl
from jax.experimental.pallas import tpu as pltpu

N_DEV = 4
SQ_SH = 256
D_MODEL = 1024
SKV = 4096
H_SH = 8
DH = 128
BLK = 64
N_CLS = 4
SCALE = 0.08838834764831843


def _take_class(a2, c, nblk):
    parts = [
        a2[(c + N_CLS * t) * BLK:(c + N_CLS * t + 1) * BLK, :]
        for t in range(nblk // N_CLS)
    ]
    return jnp.concatenate(parts, axis=0)


def kernel(x, Wq, K_ext, V_ext, Wo):
    x2 = x.reshape(SQ_SH, D_MODEL)
    K3 = K_ext.reshape(SKV, 32, DH)
    V3 = V_ext.reshape(SKV, 32, DH)

    def body(x_ref, wq_ref, k_hbm, v_hbm, wo_ref, out_ref,
             xg_ref, q_ref, ctx_ref, p_ref, k_raw, v_raw, rs_comm,
             ag_send, ag_recv, rs_send, rs_recv, kv_sems):
        my = lax.axis_index("i")
        right = lax.rem(my + 1, N_DEV)

        kcp = pltpu.make_async_copy(
            k_hbm.at[:, pl.ds(my * H_SH, H_SH), :], k_raw, kv_sems.at[0])
        vcp = pltpu.make_async_copy(
            v_hbm.at[:, pl.ds(my * H_SH, H_SH), :], v_raw, kv_sems.at[1])
        kcp.start()
        vcp.start()

        xg_ref[pl.ds(my * SQ_SH, SQ_SH), :] = x_ref[:, :]

        for h in range(N_DEV - 1):
            org = lax.rem(my + N_DEV - h, N_DEV)
            off = org * SQ_SH
            rdma = pltpu.make_async_remote_copy(
                src_ref=xg_ref.at[pl.ds(off, SQ_SH), :],
                dst_ref=xg_ref.at[pl.ds(off, SQ_SH), :],
                send_sem=ag_send.at[h],
                recv_sem=ag_recv.at[h],
                device_id=(right,),
                device_id_type=pl.DeviceIdType.MESH,
            )
            rdma.start()
            rdma.wait()

        q_ref[:, :] = jnp.dot(
            xg_ref[:, :], wq_ref[:, :], preferred_element_type=jnp.float32)

        kcp.wait()
        vcp.wait()

        for h in range(H_SH):
            kh = k_raw[:, h, :]
            vh = v_raw[:, h, :]
            qh = q_ref[:, h * DH:(h + 1) * DH]
            for c in range(N_CLS):
                qc = _take_class(qh, c, 16)
                kc = _take_class(kh, c, 64)
                vc = _take_class(vh, c, 64)
                s = jnp.dot(qc, kc.T, preferred_element_type=jnp.float32)
                s = s * SCALE
                m = jnp.max(s, axis=-1, keepdims=True)
                w = jnp.exp(s - m)
                w = w / jnp.sum(w, axis=-1, keepdims=True)
                cc = jnp.dot(w, vc, preferred_element_type=jnp.float32)
                for t in range(N_CLS):
                    b = c + N_CLS * t
                    ctx_ref[b * BLK:(b + 1) * BLK, h * DH:(h + 1) * DH] = (
                        cc[t * BLK:(t + 1) * BLK, :])

        p_ref[:, :] = jnp.dot(
            ctx_ref[:, :], wo_ref[:, :], preferred_element_type=jnp.float32)

        for sstep in range(N_DEV - 1):
            sidx = lax.rem(my + N_DEV - 1 - sstep, N_DEV)
            rdma = pltpu.make_async_remote_copy(
                src_ref=p_ref.at[pl.ds(sidx * SQ_SH, SQ_SH), :],
                dst_ref=rs_comm.at[sstep],
                send_sem=rs_send.at[sstep],
                recv_sem=rs_recv.at[sstep],
                device_id=(right,),
                device_id_type=pl.DeviceIdType.MESH,
            )
            rdma.start()
            rdma.wait()
            ridx = lax.rem(my + N_DEV - 2 - sstep, N_DEV)
            roff = ridx * SQ_SH
            p_ref[pl.ds(roff, SQ_SH), :] = (
                p_ref[pl.ds(roff, SQ_SH), :] + rs_comm[sstep])

        out_ref[:, :] = p_ref[pl.ds(my * SQ_SH, SQ_SH), :]

    out2 = pl.pallas_call(
        body,
        out_shape=jax.ShapeDtypeStruct((SQ_SH, D_MODEL), jnp.float32),
        in_specs=[
            pl.BlockSpec(memory_space=pltpu.VMEM),
            pl.BlockSpec(memory_space=pltpu.VMEM),
            pl.BlockSpec(memory_space=pltpu.ANY),
            pl.BlockSpec(memory_space=pltpu.ANY),
            pl.BlockSpec(memory_space=pltpu.VMEM),
        ],
        out_specs=pl.BlockSpec(memory_space=pltpu.VMEM),
        scratch_shapes=[
            pltpu.VMEM((N_DEV * SQ_SH, D_MODEL), jnp.float32),
            pltpu.VMEM((N_DEV * SQ_SH, H_SH * DH), jnp.float32),
            pltpu.VMEM((N_DEV * SQ_SH, H_SH * DH), jnp.float32),
            pltpu.VMEM((N_DEV * SQ_SH, D_MODEL), jnp.float32),
            pltpu.VMEM((SKV, H_SH, DH), jnp.float32),
            pltpu.VMEM((SKV, H_SH, DH), jnp.float32),
            pltpu.VMEM((N_DEV - 1, SQ_SH, D_MODEL), jnp.float32),
            pltpu.SemaphoreType.DMA((N_DEV - 1,)),
            pltpu.SemaphoreType.DMA((N_DEV - 1,)),
            pltpu.SemaphoreType.DMA((N_DEV - 1,)),
            pltpu.SemaphoreType.DMA((N_DEV - 1,)),
            pltpu.SemaphoreType.DMA((2,)),
        ],
        compiler_params=pltpu.CompilerParams(
            collective_id=0,
            vmem_limit_bytes=128 * 1024 * 1024,
        ),
    )(x2, Wq, K3, V3, Wo)

    return out2.reshape(1, SQ_SH, D_MODEL)


# baseline (device time: 126512 ns/iter reference)
import jax
import jax.numpy as jnp
from jax import lax
from jax.experimental import pallas as pl
from jax.experimental.pallas import tpu as pltpu

N_DEV = 4
SQ_SH = 256
D_MODEL = 1024
SKV = 4096
H_SH = 8
DH = 128
BLK = 64
N_CLS = 4
SCALE = 0.08838834764831843


def _take_class(a2, c, nblk):
    parts = [
        a2[(c + N_CLS * t) * BLK:(c + N_CLS * t + 1) * BLK, :]
        for t in range(nblk // N_CLS)
    ]
    return jnp.concatenate(parts, axis=0)


def kernel(x, Wq, K_ext, V_ext, Wo):
    x2 = x.reshape(SQ_SH, D_MODEL)
    K3 = K_ext.reshape(SKV, 32, DH)
    V3 = V_ext.reshape(SKV, 32, DH)

    def body(x_ref, wq_ref, k_hbm, v_hbm, wo_ref, out_ref,
             xg_ref, q_ref, ctx_ref, p_ref, k_raw, v_raw, rs_comm,
             ag_send, ag_recv, rs_send, rs_recv, kv_sems):
        my = lax.axis_index("i")
        right = lax.rem(my + 1, N_DEV)

        kcp = pltpu.make_async_copy(
            k_hbm.at[:, pl.ds(my * H_SH, H_SH), :], k_raw, kv_sems.at[0])
        vcp = pltpu.make_async_copy(
            v_hbm.at[:, pl.ds(my * H_SH, H_SH), :], v_raw, kv_sems.at[1])
        kcp.start()
        vcp.start()

        xg_ref[pl.ds(my * SQ_SH, SQ_SH), :] = x_ref[:, :]

        for h in range(N_DEV - 1):
            org = lax.rem(my + N_DEV - h, N_DEV)
            off = org * SQ_SH
            rdma = pltpu.make_async_remote_copy(
                src_ref=xg_ref.at[pl.ds(off, SQ_SH), :],
                dst_ref=xg_ref.at[pl.ds(off, SQ_SH), :],
                send_sem=ag_send.at[h],
                recv_sem=ag_recv.at[h],
                device_id=(right,),
                device_id_type=pl.DeviceIdType.MESH,
            )
            rdma.start()
            rdma.wait()

        q_ref[:, :] = jnp.dot(
            xg_ref[:, :], wq_ref[:, :], preferred_element_type=jnp.float32)

        kcp.wait()
        vcp.wait()

        for h in range(H_SH):
            kh = k_raw[:, h, :]
            vh = v_raw[:, h, :]
            qh = q_ref[:, h * DH:(h + 1) * DH]
            for c in range(N_CLS):
                qc = _take_class(qh, c, 16)
                kc = _take_class(kh, c, 64)
                vc = _take_class(vh, c, 64)
                s = jnp.dot(qc, kc.T, preferred_element_type=jnp.float32)
                s = s * SCALE
                m = jnp.max(s, axis=-1, keepdims=True)
                w = jnp.exp(s - m)
                w = w / jnp.sum(w, axis=-1, keepdims=True)
                cc = jnp.dot(w, vc, preferred_element_type=jnp.float32)
                for t in range(N_CLS):
                    b = c + N_CLS * t
                    ctx_ref[b * BLK:(b + 1) * BLK, h * DH:(h + 1) * DH] = (
                        cc[t * BLK:(t + 1) * BLK, :])

        p_ref[:, :] = jnp.dot(
            ctx_ref[:, :], wo_ref[:, :], preferred_element_type=jnp.float32)

        for sstep in range(N_DEV - 1):
            sidx = lax.rem(my + N_DEV - 1 - sstep, N_DEV)
            rdma = pltpu.make_async_remote_copy(
                src_ref=p_ref.at[pl.ds(sidx * SQ_SH, SQ_SH), :],
                dst_ref=rs_comm.at[sstep],
                send_sem=rs_send.at[sstep],
                recv_sem=rs_recv.at[sstep],
                device_id=(right,),
                device_id_type=pl.DeviceIdType.MESH,
            )
            rdma.start()
            rdma.wait()
            ridx = lax.rem(my + N_DEV - 2 - sstep, N_DEV)
            roff = ridx * SQ_SH
            p_ref[pl.ds(roff, SQ_SH), :] = (
                p_ref[pl.ds(roff, SQ_SH), :] + rs_comm[sstep])

        out_ref[:, :] = p_ref[pl.ds(my * SQ_SH, SQ_SH), :]

    out2 = pl.pallas_call(
        body,
        out_shape=jax.ShapeDtypeStruct((SQ_SH, D_MODEL), jnp.float32),
        in_specs=[
            pl.BlockSpec(memory_space=pltpu.VMEM),
            pl.BlockSpec(memory_space=pltpu.VMEM),
            pl.BlockSpec(memory_space=pl.ANY),
            pl.BlockSpec(memory_space=pl.ANY),
            pl.BlockSpec(memory_space=pltpu.VMEM),
        ],
        out_specs=pl.BlockSpec(memory_space=pltpu.VMEM),
        scratch_shapes=[
            pltpu.VMEM((N_DEV * SQ_SH, D_MODEL), jnp.float32),
            pltpu.VMEM((N_DEV * SQ_SH, H_SH * DH), jnp.float32),
            pltpu.VMEM((N_DEV * SQ_SH, H_SH * DH), jnp.float32),
            pltpu.VMEM((N_DEV * SQ_SH, D_MODEL), jnp.float32),
            pltpu.VMEM((SKV, H_SH, DH), jnp.float32),
            pltpu.VMEM((SKV, H_SH, DH), jnp.float32),
            pltpu.VMEM((N_DEV - 1, SQ_SH, D_MODEL), jnp.float32),
            pltpu.SemaphoreType.DMA((N_DEV - 1,)),
            pltpu.SemaphoreType.DMA((N_DEV - 1,)),
            pltpu.SemaphoreType.DMA((N_DEV - 1,)),
            pltpu.SemaphoreType.DMA((N_DEV - 1,)),
            pltpu.SemaphoreType.DMA((2,)),
        ],
        compiler_params=pltpu.CompilerParams(
            vmem_limit_bytes=128 * 1024 * 1024,
        ),
    )(x2, Wq, K3, V3, Wo)

    return out2.reshape(1, SQ_SH, D_MODEL)


# device time: 109499 ns/iter; 1.1554x vs baseline; 1.1554x over previous
import jax
import jax.numpy as jnp
from jax import lax
from jax.experimental import pallas as pl
from jax.experimental.pallas import tpu as pltpu

N_DEV = 4
SQ_SH = 256
D_MODEL = 1024
SKV = 4096
H_SH = 8
H_TOT = 32
DH = 128
BLK = 64
N_CLS = 4
NT = SKV // BLK // N_CLS
KV_CLS = NT * BLK
SCALE = 0.08838834764831843
F32 = jnp.float32


def kernel(x, Wq, K_ext, V_ext, Wo):
    x2 = x.reshape(SQ_SH, D_MODEL)
    K5 = K_ext.reshape(NT, N_CLS, BLK, H_TOT, DH)
    V5 = V_ext.reshape(NT, N_CLS, BLK, H_TOT, DH)

    def body(x_ref, wq_ref, k_hbm, v_hbm, wo_ref, out_ref,
             xg_ref, q_ref, ctx_ref, p_ref, k_raw, v_raw, rs_comm,
             ag_send, ag_recv, rs_send, rs_recv, kv_sems):
        my = lax.axis_index("i")
        right = lax.rem(my + 1, N_DEV)
        offs = [lax.rem(my + N_DEV - s, N_DEV) * SQ_SH for s in range(N_DEV)]

        kv_copies = []
        for h in range(H_SH):
            gh = my * H_SH + h
            for c in range(N_CLS):
                kcp = pltpu.make_async_copy(
                    k_hbm.at[:, c, :, gh, :], k_raw.at[h, c], kv_sems.at[0])
                vcp = pltpu.make_async_copy(
                    v_hbm.at[:, c, :, gh, :], v_raw.at[h, c], kv_sems.at[1])
                kcp.start()
                vcp.start()
                kv_copies += [kcp, vcp]

        ag = []

        def ag_start(s):
            r = pltpu.make_async_remote_copy(
                src_ref=xg_ref.at[pl.ds(offs[s], SQ_SH), :],
                dst_ref=xg_ref.at[pl.ds(offs[s], SQ_SH), :],
                send_sem=ag_send.at[s],
                recv_sem=ag_recv.at[s],
                device_id=(right,),
                device_id_type=pl.DeviceIdType.MESH,
            )
            r.start()
            ag.append(r)

        rs = []

        def rs_start(s, src_off):
            r = pltpu.make_async_remote_copy(
                src_ref=p_ref.at[pl.ds(src_off, SQ_SH), :],
                dst_ref=rs_comm.at[s],
                send_sem=rs_send.at[s],
                recv_sem=rs_recv.at[s],
                device_id=(right,),
                device_id_type=pl.DeviceIdType.MESH,
            )
            r.start()
            rs.append(r)

        def q_chunk(j_off):
            q_ref[pl.ds(j_off, SQ_SH), :] = jnp.dot(
                xg_ref[pl.ds(j_off, SQ_SH), :], wq_ref[:, :],
                preferred_element_type=F32)

        def attn_p_chunk(j_off):
            for h in range(H_SH):
                for u in range(N_CLS):
                    qb = q_ref[pl.ds(j_off + u * BLK, BLK),
                               h * DH:(h + 1) * DH]
                    kc = k_raw[h, u].reshape(KV_CLS, DH)
                    vc = v_raw[h, u].reshape(KV_CLS, DH)
                    s = lax.dot_general(
                        qb, kc, (((1,), (1,)), ((), ())),
                        preferred_element_type=F32) * SCALE
                    m = jnp.max(s, axis=-1, keepdims=True)
                    w = jnp.exp(s - m)
                    w = w / jnp.sum(w, axis=-1, keepdims=True)
                    cc = jnp.dot(w, vc, preferred_element_type=F32)
                    ctx_ref[pl.ds(j_off + u * BLK, BLK),
                            h * DH:(h + 1) * DH] = cc
            p_ref[pl.ds(j_off, SQ_SH), :] = jnp.dot(
                ctx_ref[pl.ds(j_off, SQ_SH), :], wo_ref[:, :],
                preferred_element_type=F32)

        xg_ref[pl.ds(offs[0], SQ_SH), :] = x_ref[:, :]
        ag_start(0)
        q_chunk(offs[0])
        for cp in kv_copies:
            cp.wait()
        attn_p_chunk(offs[0])

        ag[0].wait_recv()
        ag_start(1)
        q_chunk(offs[1])
        attn_p_chunk(offs[1])
        rs_start(0, offs[1])

        ag[1].wait_recv()
        ag_start(2)
        q_chunk(offs[2])
        attn_p_chunk(offs[2])
        rs[0].wait_recv()
        p_ref[pl.ds(offs[2], SQ_SH), :] = (
            p_ref[pl.ds(offs[2], SQ_SH), :] + rs_comm[0])
        rs_start(1, offs[2])

        ag[2].wait_recv()
        q_chunk(offs[3])
        attn_p_chunk(offs[3])
        rs[1].wait_recv()
        p_ref[pl.ds(offs[3], SQ_SH), :] = (
            p_ref[pl.ds(offs[3], SQ_SH), :] + rs_comm[1])
        rs_start(2, offs[3])

        rs[2].wait_recv()
        out_ref[:, :] = p_ref[pl.ds(offs[0], SQ_SH), :] + rs_comm[2]

        for r in ag:
            r.wait_send()
        for r in rs:
            r.wait_send()

    out2 = pl.pallas_call(
        body,
        out_shape=jax.ShapeDtypeStruct((SQ_SH, D_MODEL), jnp.float32),
        in_specs=[
            pl.BlockSpec(memory_space=pltpu.VMEM),
            pl.BlockSpec(memory_space=pltpu.VMEM),
            pl.BlockSpec(memory_space=pl.ANY),
            pl.BlockSpec(memory_space=pl.ANY),
            pl.BlockSpec(memory_space=pltpu.VMEM),
        ],
        out_specs=pl.BlockSpec(memory_space=pltpu.VMEM),
        scratch_shapes=[
            pltpu.VMEM((N_DEV * SQ_SH, D_MODEL), F32),
            pltpu.VMEM((N_DEV * SQ_SH, H_SH * DH), F32),
            pltpu.VMEM((N_DEV * SQ_SH, H_SH * DH), F32),
            pltpu.VMEM((N_DEV * SQ_SH, D_MODEL), F32),
            pltpu.VMEM((H_SH, N_CLS, NT, BLK, DH), F32),
            pltpu.VMEM((H_SH, N_CLS, NT, BLK, DH), F32),
            pltpu.VMEM((N_DEV - 1, SQ_SH, D_MODEL), F32),
            pltpu.SemaphoreType.DMA((N_DEV - 1,)),
            pltpu.SemaphoreType.DMA((N_DEV - 1,)),
            pltpu.SemaphoreType.DMA((N_DEV - 1,)),
            pltpu.SemaphoreType.DMA((N_DEV - 1,)),
            pltpu.SemaphoreType.DMA((2,)),
        ],
        compiler_params=pltpu.CompilerParams(
            vmem_limit_bytes=128 * 1024 * 1024,
        ),
    )(x2, Wq, K5, V5, Wo)

    return out2.reshape(1, SQ_SH, D_MODEL)


# device time: 56381 ns/iter; 2.2439x vs baseline; 1.9421x over previous
import jax
import jax.numpy as jnp
from jax import lax
from jax.experimental import pallas as pl
from jax.experimental.pallas import tpu as pltpu

N_DEV = 4
SQ_SH = 256
D_MODEL = 1024
SKV = 4096
H_SH = 8
H_TOT = 32
DH = 128
BLK = 64
N_CLS = 4
NT = SKV // BLK // N_CLS
KV_CLS = NT * BLK
SCALE = 0.08838834764831843
F32 = jnp.float32
BF16 = jnp.bfloat16


def kernel(x, Wq, K_ext, V_ext, Wo):
    x2 = x.reshape(SQ_SH, D_MODEL)
    K5 = K_ext.reshape(NT, N_CLS, BLK, H_TOT, DH)
    V5 = V_ext.reshape(NT, N_CLS, BLK, H_TOT, DH)

    def body(x_ref, wq_ref, k_hbm, v_hbm, wo_ref, out_ref,
             xg_ref, q_ref, ctx_ref, p_ref, kb_ref, vb_ref,
             kst, vst, rs_comm, wq_bf, wo_bf, pb_ref,
             ag_send, ag_recv, rs_send, rs_recv, kv_sems):
        my = lax.axis_index("i")
        right = lax.rem(my + 1, N_DEV)
        left = lax.rem(my + N_DEV - 1, N_DEV)
        offs = [lax.rem(my + N_DEV - s, N_DEV) * SQ_SH for s in range(N_DEV)]

        barrier = pltpu.get_barrier_semaphore()
        for nbr in (left, right):
            pl.semaphore_signal(
                barrier, inc=1, device_id=(nbr,),
                device_id_type=pl.DeviceIdType.MESH)
        pl.semaphore_wait(barrier, 2)

        kv_dmas = {}

        def kv_issue(h):
            gh = my * H_SH + h
            slot = h % 2
            ds = []
            for c in range(N_CLS):
                kcp = pltpu.make_async_copy(
                    k_hbm.at[:, c, :, gh, :], kst.at[slot, c],
                    kv_sems.at[0, slot])
                vcp = pltpu.make_async_copy(
                    v_hbm.at[:, c, :, gh, :], vst.at[slot, c],
                    kv_sems.at[1, slot])
                kcp.start()
                vcp.start()
                ds += [kcp, vcp]
            kv_dmas[h] = ds

        def kv_land(h):
            for d in kv_dmas[h]:
                d.wait()
            kb_ref[h, :, :, :, :] = kst[h % 2].astype(BF16)
            vb_ref[h, :, :, :, :] = vst[h % 2].astype(BF16)
            if h + 2 < H_SH:
                kv_issue(h + 2)

        ag = []

        def ag_start(s):
            r = pltpu.make_async_remote_copy(
                src_ref=xg_ref.at[pl.ds(offs[s], SQ_SH), :],
                dst_ref=xg_ref.at[pl.ds(offs[s], SQ_SH), :],
                send_sem=ag_send.at[s],
                recv_sem=ag_recv.at[s],
                device_id=(right,),
                device_id_type=pl.DeviceIdType.MESH,
            )
            r.start()
            ag.append(r)

        rs = []

        def rs_start(s, src_off):
            pb_ref[s, :, :] = p_ref[pl.ds(src_off, SQ_SH), :].astype(BF16)
            r = pltpu.make_async_remote_copy(
                src_ref=pb_ref.at[s],
                dst_ref=rs_comm.at[s],
                send_sem=rs_send.at[s],
                recv_sem=rs_recv.at[s],
                device_id=(right,),
                device_id_type=pl.DeviceIdType.MESH,
            )
            r.start()
            rs.append(r)

        def q_chunk(j_off):
            q_ref[pl.ds(j_off, SQ_SH), :] = (
                jnp.dot(xg_ref[pl.ds(j_off, SQ_SH), :], wq_bf[:, :],
                        preferred_element_type=F32) * SCALE).astype(BF16)

        def attn_head(j_off, h):
            qb3 = q_ref[pl.ds(j_off, SQ_SH),
                        h * DH:(h + 1) * DH].reshape(N_CLS, BLK, DH)
            kc3 = kb_ref[h].reshape(N_CLS, KV_CLS, DH)
            vc3 = vb_ref[h].reshape(N_CLS, KV_CLS, DH)
            s = lax.dot_general(
                qb3, kc3, (((2,), (2,)), ((0,), (0,))),
                preferred_element_type=F32)
            w = jnp.exp(s)
            w = (w * (1.0 / jnp.sum(w, axis=-1, keepdims=True))
                 ).astype(BF16)
            cc = lax.dot_general(
                w, vc3, (((2,), (1,)), ((0,), (0,))),
                preferred_element_type=F32)
            ctx_ref[pl.ds(j_off, SQ_SH), h * DH:(h + 1) * DH] = (
                cc.reshape(SQ_SH, DH).astype(BF16))

        def oproj_chunk(j_off):
            p_ref[pl.ds(j_off, SQ_SH), :] = jnp.dot(
                ctx_ref[pl.ds(j_off, SQ_SH), :], wo_bf[:, :],
                preferred_element_type=F32)

        def attn_p_chunk(j_off):
            for h in range(H_SH):
                attn_head(j_off, h)
            oproj_chunk(j_off)

        kv_issue(0)
        kv_issue(1)
        xg_ref[pl.ds(offs[0], SQ_SH), :] = x_ref[:, :].astype(BF16)
        ag_start(0)
        wq_bf[:, :] = wq_ref[:, :].astype(BF16)
        wo_bf[:, :] = wo_ref[:, :].astype(BF16)
        q_chunk(offs[0])

        ag[0].wait_recv()
        ag_start(1)
        q_chunk(offs[1])
        for h in range(H_SH):
            kv_land(h)
            attn_head(offs[1], h)
        oproj_chunk(offs[1])
        rs_start(0, offs[1])

        ag[1].wait_recv()
        ag_start(2)
        q_chunk(offs[2])
        attn_p_chunk(offs[2])
        rs[0].wait_recv()
        p_ref[pl.ds(offs[2], SQ_SH), :] = (
            p_ref[pl.ds(offs[2], SQ_SH), :] + rs_comm[0].astype(F32))
        rs_start(1, offs[2])

        ag[2].wait_recv()
        q_chunk(offs[3])
        attn_p_chunk(offs[3])
        rs[1].wait_recv()
        p_ref[pl.ds(offs[3], SQ_SH), :] = (
            p_ref[pl.ds(offs[3], SQ_SH), :] + rs_comm[1].astype(F32))
        rs_start(2, offs[3])

        attn_p_chunk(offs[0])
        rs[2].wait_recv()
        out_ref[:, :] = (
            p_ref[pl.ds(offs[0], SQ_SH), :] + rs_comm[2].astype(F32))

        for r in ag:
            r.wait_send()
        for r in rs:
            r.wait_send()

    out2 = pl.pallas_call(
        body,
        out_shape=jax.ShapeDtypeStruct((SQ_SH, D_MODEL), jnp.float32),
        in_specs=[
            pl.BlockSpec(memory_space=pltpu.VMEM),
            pl.BlockSpec(memory_space=pltpu.VMEM),
            pl.BlockSpec(memory_space=pl.ANY),
            pl.BlockSpec(memory_space=pl.ANY),
            pl.BlockSpec(memory_space=pltpu.VMEM),
        ],
        out_specs=pl.BlockSpec(memory_space=pltpu.VMEM),
        scratch_shapes=[
            pltpu.VMEM((N_DEV * SQ_SH, D_MODEL), BF16),
            pltpu.VMEM((N_DEV * SQ_SH, H_SH * DH), BF16),
            pltpu.VMEM((N_DEV * SQ_SH, H_SH * DH), BF16),
            pltpu.VMEM((N_DEV * SQ_SH, D_MODEL), F32),
            pltpu.VMEM((H_SH, N_CLS, NT, BLK, DH), BF16),
            pltpu.VMEM((H_SH, N_CLS, NT, BLK, DH), BF16),
            pltpu.VMEM((2, N_CLS, NT, BLK, DH), F32),
            pltpu.VMEM((2, N_CLS, NT, BLK, DH), F32),
            pltpu.VMEM((N_DEV - 1, SQ_SH, D_MODEL), BF16),
            pltpu.VMEM((D_MODEL, H_SH * DH), BF16),
            pltpu.VMEM((H_SH * DH, D_MODEL), BF16),
            pltpu.VMEM((N_DEV - 1, SQ_SH, D_MODEL), BF16),
            pltpu.SemaphoreType.DMA((N_DEV - 1,)),
            pltpu.SemaphoreType.DMA((N_DEV - 1,)),
            pltpu.SemaphoreType.DMA((N_DEV - 1,)),
            pltpu.SemaphoreType.DMA((N_DEV - 1,)),
            pltpu.SemaphoreType.DMA((2, 2)),
        ],
        compiler_params=pltpu.CompilerParams(
            collective_id=0,
            vmem_limit_bytes=128 * 1024 * 1024,
        ),
    )(x2, Wq, K5, V5, Wo)

    return out2.reshape(1, SQ_SH, D_MODEL)
